# baseline (speedup 1.0000x reference)
import functools

import jax
import jax.numpy as jnp
from jax import lax
from jax.experimental import pallas as pl
from jax.experimental.pallas import tpu as pltpu
from jax.experimental.pallas import tpu_sc as plsc

N_USERS = 50000
N_NODES = 100000
HALF = 50000
EMB = 32
E = 1600000

NC = 2
NS = 16
B = 768
CH = 128
NCH = B // CH
NBLK = 131
EPT = NBLK * B
E_PAD = EPT * NS
ACC_ROWS = 50048
DUMMY = 50000
ZROWS = ACC_ROWS // NS
WROWS = 3128

_GDN = lax.GatherDimensionNumbers(
    offset_dims=(), collapsed_slice_dims=(0,), start_index_map=(0,))


def _bcast_lane(v, l):
    idx = jnp.full((16, 1), l, jnp.int32)
    return lax.gather(v, idx, dimension_numbers=_GDN, slice_sizes=(1,),
                      mode=lax.GatherScatterMode.PROMISE_IN_BOUNDS)


_mesh = functools.partial(
    plsc.VectorSubcoreMesh, core_axis_name="c", subcore_axis_name="s",
    num_cores=NC, num_subcores=NS)


def _layer_body(x_hbm, col_hbm, row_hbm, val_hbm, y_hbm,
                col_v, row_v, val_v, lidx_v, rows_v, acc, sem_g, sem_s):
    c = lax.axis_index("c")
    s = lax.axis_index("s")
    base_node = c * HALF

    zvec = jnp.zeros((16,), jnp.float32)

    def zero_rows(t, _):
        rows_v[t, pl.ds(0, 16)] = zvec
        rows_v[t, pl.ds(16, 16)] = zvec
        return 0

    lax.fori_loop(0, B, zero_rows, 0)
    zb = s * ZROWS

    def zero_acc(k, _):
        pltpu.sync_copy(rows_v, acc.at[pl.ds(zb + k * B, B)])
        return 0

    lax.fori_loop(0, ZROWS // B, zero_acc, 0)
    ztail = ZROWS % B
    pltpu.sync_copy(rows_v.at[pl.ds(0, ztail)],
                    acc.at[pl.ds(zb + ZROWS - ztail, ztail)])
    plsc.subcore_barrier()

    def block(b, _):
        rb = s * (EPT // CH) + b * NCH
        eb = s * EPT + b * B
        pltpu.sync_copy(col_hbm.at[pl.ds(rb, NCH)], col_v)
        pltpu.sync_copy(row_hbm.at[pl.ds(eb, B)], row_v)
        pltpu.sync_copy(val_hbm.at[pl.ds(eb, B)], val_v)

        descs = [
            pltpu.async_copy(x_hbm.at[col_v.at[j]],
                             rows_v.at[pl.ds(j * CH, CH)], sem_g)
            for j in range(NCH)
        ]

        for d in descs:
            d.wait()

        def lidx(t, _):
            r = row_v[pl.ds(t * 16, 16)]
            ok = (r >= base_node) & (r < base_node + HALF)
            lidx_v[t >> 3, pl.ds((t & 7) * 16, 16)] = jnp.where(
                ok, r - base_node, DUMMY)
            return 0

        lax.fori_loop(0, B // 16, lidx, 0)

        def scale(g, _):
            v = val_v[pl.ds(g * 16, 16)]
            e0 = g * 16
            for l in range(16):
                vb = _bcast_lane(v, l)
                e = e0 + l
                r0 = rows_v[e, pl.ds(0, 16)]
                rows_v[e, pl.ds(0, 16)] = r0 * vb
                r1 = rows_v[e, pl.ds(16, 16)]
                rows_v[e, pl.ds(16, 16)] = r1 * vb
            return 0

        lax.fori_loop(0, B // 16, scale, 0)

        descs2 = [
            pltpu.async_copy(rows_v.at[pl.ds(j * CH, CH)],
                             acc.at[lidx_v.at[j]], sem_s, add=True)
            for j in range(NCH)
        ]
        for d in descs2:
            d.wait()
        return 0

    lax.fori_loop(0, NBLK, block, 0)
    plsc.subcore_barrier()

    g0 = s * WROWS

    def wb(k, _):
        pltpu.sync_copy(acc.at[pl.ds(g0 + k * B, B)], rows_v)
        pltpu.sync_copy(rows_v, y_hbm.at[pl.ds(base_node + g0 + k * B, B)])
        return 0

    lax.fori_loop(0, WROWS // B, wb, 0)
    done = (WROWS // B) * B

    @pl.when(s < NS - 1)
    def _():
        rem = WROWS - done
        pltpu.sync_copy(acc.at[pl.ds(g0 + done, rem)],
                        rows_v.at[pl.ds(0, rem)])
        pltpu.sync_copy(rows_v.at[pl.ds(0, rem)],
                        y_hbm.at[pl.ds(base_node + g0 + done, rem)])

    @pl.when(s == NS - 1)
    def _():
        rem = HALF - (NS - 1) * WROWS - done
        pltpu.sync_copy(acc.at[pl.ds(g0 + done, rem)],
                        rows_v.at[pl.ds(0, rem)])
        pltpu.sync_copy(rows_v.at[pl.ds(0, rem)],
                        y_hbm.at[pl.ds(base_node + g0 + done, rem)])


_layer = pl.kernel(
    _layer_body,
    out_type=jax.ShapeDtypeStruct((N_NODES, EMB), jnp.float32),
    mesh=_mesh(),
    compiler_params=pltpu.CompilerParams(use_tc_tiling_on_sc=False),
    scratch_types=[
        pltpu.VMEM((NCH, CH), jnp.int32),
        pltpu.VMEM((B,), jnp.int32),
        pltpu.VMEM((B,), jnp.float32),
        pltpu.VMEM((NCH, CH), jnp.int32),
        pltpu.VMEM((B, EMB), jnp.float32),
        pltpu.VMEM_SHARED((ACC_ROWS, EMB), jnp.float32),
        pltpu.SemaphoreType.DMA,
        pltpu.SemaphoreType.DMA,
    ],
)

MCH = 1024


def _mean_body(y1_hbm, y2_hbm, y3_hbm, out_hbm, a_v, b_v, c_v):
    c = lax.axis_index("c")
    s = lax.axis_index("s")
    base = c * HALF + s * WROWS
    third = jnp.float32(1.0 / 3.0)

    def chunk(rb, n):
        pltpu.sync_copy(y1_hbm.at[pl.ds(rb, n)], a_v.at[pl.ds(0, n)])
        pltpu.sync_copy(y2_hbm.at[pl.ds(rb, n)], b_v.at[pl.ds(0, n)])
        pltpu.sync_copy(y3_hbm.at[pl.ds(rb, n)], c_v.at[pl.ds(0, n)])

        def avg(t, _):
            i = t >> 1
            o = (t & 1) * 16
            v = (a_v[i, pl.ds(o, 16)] + b_v[i, pl.ds(o, 16)]
                 + c_v[i, pl.ds(o, 16)])
            a_v[i, pl.ds(o, 16)] = v * third
            return 0

        lax.fori_loop(0, n * 2, avg, 0)
        pltpu.sync_copy(a_v.at[pl.ds(0, n)], out_hbm.at[pl.ds(rb, n)])

    for k in range(3):
        chunk(base + k * MCH, MCH)

    @pl.when(s < NS - 1)
    def _():
        chunk(base + 3 * MCH, WROWS - 3 * MCH)

    @pl.when(s == NS - 1)
    def _():
        chunk(base + 3 * MCH, HALF - (NS - 1) * WROWS - 3 * MCH)


_mean = pl.kernel(
    _mean_body,
    out_type=jax.ShapeDtypeStruct((N_NODES, EMB), jnp.float32),
    mesh=_mesh(),
    compiler_params=pltpu.CompilerParams(use_tc_tiling_on_sc=False),
    scratch_types=[
        pltpu.VMEM((MCH, EMB), jnp.float32),
        pltpu.VMEM((MCH, EMB), jnp.float32),
        pltpu.VMEM((MCH, EMB), jnp.float32),
    ],
)


def _chk():
    assert NS * WROWS >= HALF and (NS - 1) * WROWS < HALF
    assert WROWS % 8 == 0 and ACC_ROWS == NS * ZROWS and ZROWS % 8 == 0
    assert NBLK * B * NS == E_PAD and E_PAD >= E


_chk()


@jax.jit
def kernel(ego_embeddings, adj_row, adj_col, adj_val):
    x = ego_embeddings.astype(jnp.float32)
    pad = E_PAD - E
    col = jnp.concatenate(
        [adj_col.astype(jnp.int32), jnp.zeros((pad,), jnp.int32)]
    ).reshape(E_PAD // CH, CH)
    row = jnp.concatenate(
        [adj_row.astype(jnp.int32), jnp.zeros((pad,), jnp.int32)])
    val = jnp.concatenate(
        [adj_val.astype(jnp.float32), jnp.zeros((pad,), jnp.float32)])

    y1 = _layer(x, col, row, val)
    y2 = _layer(y1, col, row, val)
    y3 = _layer(y2, col, row, val)
    m = _mean(y1, y2, y3)
    return m[:N_USERS], m[N_USERS:]


# speedup vs baseline: 1.0491x; 1.0491x over previous
import functools

import jax
import jax.numpy as jnp
from jax import lax
from jax.experimental import pallas as pl
from jax.experimental.pallas import tpu as pltpu
from jax.experimental.pallas import tpu_sc as plsc

N_USERS = 50000
N_NODES = 100000
HALF = 50000
EMB = 32
E = 1600000

NC = 2
NS = 16
B = 512
CH = 128
NCH = B // CH
NBLK = 196
EPT = NBLK * B
E_PAD = EPT * NS
ACC_ROWS = 50048
DUMMY = 50000
ZROWS = ACC_ROWS // NS
WROWS = 3128

_GDN = lax.GatherDimensionNumbers(
    offset_dims=(), collapsed_slice_dims=(0,), start_index_map=(0,))


def _bcast_lane(v, l):
    idx = jnp.full((16, 1), l, jnp.int32)
    return lax.gather(v, idx, dimension_numbers=_GDN, slice_sizes=(1,),
                      mode=lax.GatherScatterMode.PROMISE_IN_BOUNDS)


_mesh = functools.partial(
    plsc.VectorSubcoreMesh, core_axis_name="c", subcore_axis_name="s",
    num_cores=NC, num_subcores=NS)


def _layer_body(x_hbm, col_hbm, row_hbm, val_hbm, y_hbm,
                col_v, row_v, val_v, lidx_v, rows_v, acc, sem_g, sem_s):
    c = lax.axis_index("c")
    s = lax.axis_index("s")
    base_node = c * HALF

    zvec = jnp.zeros((16,), jnp.float32)

    def zero_rows(t, _):
        rows_v[t, pl.ds(0, 16)] = zvec
        rows_v[t, pl.ds(16, 16)] = zvec
        return 0

    lax.fori_loop(0, B, zero_rows, 0)
    zb = s * ZROWS

    def zero_acc(k, _):
        pltpu.sync_copy(rows_v, acc.at[pl.ds(zb + k * B, B)])
        return 0

    lax.fori_loop(0, ZROWS // B, zero_acc, 0)
    ztail = ZROWS % B
    pltpu.sync_copy(rows_v.at[pl.ds(0, ztail)],
                    acc.at[pl.ds(zb + ZROWS - ztail, ztail)])
    plsc.subcore_barrier()

    def block(b, _):
        rb = s * (EPT // CH) + b * NCH
        eb = s * EPT + b * B
        pltpu.sync_copy(col_hbm.at[pl.ds(rb, NCH)], col_v)
        pltpu.sync_copy(row_hbm.at[pl.ds(eb, B)], row_v)
        pltpu.sync_copy(val_hbm.at[pl.ds(eb, B)], val_v)

        descs = [
            pltpu.async_copy(x_hbm.at[col_v.at[j]],
                             rows_v.at[pl.ds(j * CH, CH)], sem_g)
            for j in range(NCH)
        ]

        for d in descs:
            d.wait()

        def lidx(t, _):
            r = row_v[pl.ds(t * 16, 16)]
            ok = (r >= base_node) & (r < base_node + HALF)
            lidx_v[t >> 3, pl.ds((t & 7) * 16, 16)] = jnp.where(
                ok, r - base_node, DUMMY)
            return 0

        lax.fori_loop(0, B // 16, lidx, 0)

        def scale(g, _):
            v = val_v[pl.ds(g * 16, 16)]
            e0 = g * 16
            for l in range(16):
                vb = _bcast_lane(v, l)
                e = e0 + l
                r0 = rows_v[e, pl.ds(0, 16)]
                rows_v[e, pl.ds(0, 16)] = r0 * vb
                r1 = rows_v[e, pl.ds(16, 16)]
                rows_v[e, pl.ds(16, 16)] = r1 * vb
            return 0

        lax.fori_loop(0, B // 16, scale, 0)

        descs2 = [
            pltpu.async_copy(rows_v.at[pl.ds(j * CH, CH)],
                             acc.at[lidx_v.at[j]], sem_s, add=True)
            for j in range(NCH)
        ]
        for d in descs2:
            d.wait()
        return 0

    lax.fori_loop(0, NBLK, block, 0)
    plsc.subcore_barrier()

    g0 = s * WROWS

    def wb(k, _):
        pltpu.sync_copy(acc.at[pl.ds(g0 + k * B, B)], rows_v)
        pltpu.sync_copy(rows_v, y_hbm.at[pl.ds(base_node + g0 + k * B, B)])
        return 0

    lax.fori_loop(0, WROWS // B, wb, 0)
    done = (WROWS // B) * B

    @pl.when(s < NS - 1)
    def _():
        rem = WROWS - done
        pltpu.sync_copy(acc.at[pl.ds(g0 + done, rem)],
                        rows_v.at[pl.ds(0, rem)])
        pltpu.sync_copy(rows_v.at[pl.ds(0, rem)],
                        y_hbm.at[pl.ds(base_node + g0 + done, rem)])

    @pl.when(s == NS - 1)
    def _():
        rem = HALF - (NS - 1) * WROWS - done
        pltpu.sync_copy(acc.at[pl.ds(g0 + done, rem)],
                        rows_v.at[pl.ds(0, rem)])
        pltpu.sync_copy(rows_v.at[pl.ds(0, rem)],
                        y_hbm.at[pl.ds(base_node + g0 + done, rem)])


_layer = pl.kernel(
    _layer_body,
    out_type=jax.ShapeDtypeStruct((N_NODES, EMB), jnp.float32),
    mesh=_mesh(),
    compiler_params=pltpu.CompilerParams(use_tc_tiling_on_sc=False),
    scratch_types=[
        pltpu.VMEM((NCH, CH), jnp.int32),
        pltpu.VMEM((B,), jnp.int32),
        pltpu.VMEM((B,), jnp.float32),
        pltpu.VMEM((NCH, CH), jnp.int32),
        pltpu.VMEM((B, EMB), jnp.float32),
        pltpu.VMEM_SHARED((ACC_ROWS, EMB), jnp.float32),
        pltpu.SemaphoreType.DMA,
        pltpu.SemaphoreType.DMA,
    ],
)

MCH = 1024


def _mean_body(y1_hbm, y2_hbm, y3_hbm, out_hbm, a_v, b_v, c_v):
    c = lax.axis_index("c")
    s = lax.axis_index("s")
    base = c * HALF + s * WROWS
    third = jnp.float32(1.0 / 3.0)

    def chunk(rb, n):
        pltpu.sync_copy(y1_hbm.at[pl.ds(rb, n)], a_v.at[pl.ds(0, n)])
        pltpu.sync_copy(y2_hbm.at[pl.ds(rb, n)], b_v.at[pl.ds(0, n)])
        pltpu.sync_copy(y3_hbm.at[pl.ds(rb, n)], c_v.at[pl.ds(0, n)])

        def avg(t, _):
            i = t >> 1
            o = (t & 1) * 16
            v = (a_v[i, pl.ds(o, 16)] + b_v[i, pl.ds(o, 16)]
                 + c_v[i, pl.ds(o, 16)])
            a_v[i, pl.ds(o, 16)] = v * third
            return 0

        lax.fori_loop(0, n * 2, avg, 0)
        pltpu.sync_copy(a_v.at[pl.ds(0, n)], out_hbm.at[pl.ds(rb, n)])

    for k in range(3):
        chunk(base + k * MCH, MCH)

    @pl.when(s < NS - 1)
    def _():
        chunk(base + 3 * MCH, WROWS - 3 * MCH)

    @pl.when(s == NS - 1)
    def _():
        chunk(base + 3 * MCH, HALF - (NS - 1) * WROWS - 3 * MCH)


_mean = pl.kernel(
    _mean_body,
    out_type=jax.ShapeDtypeStruct((N_NODES, EMB), jnp.float32),
    mesh=_mesh(),
    compiler_params=pltpu.CompilerParams(use_tc_tiling_on_sc=False),
    scratch_types=[
        pltpu.VMEM((MCH, EMB), jnp.float32),
        pltpu.VMEM((MCH, EMB), jnp.float32),
        pltpu.VMEM((MCH, EMB), jnp.float32),
    ],
)


def _chk():
    assert NS * WROWS >= HALF and (NS - 1) * WROWS < HALF
    assert WROWS % 8 == 0 and ACC_ROWS == NS * ZROWS and ZROWS % 8 == 0
    assert NBLK * B * NS == E_PAD and E_PAD >= E


_chk()


@jax.jit
def kernel(ego_embeddings, adj_row, adj_col, adj_val):
    x = ego_embeddings.astype(jnp.float32)
    pad = E_PAD - E
    col = jnp.concatenate(
        [adj_col.astype(jnp.int32), jnp.zeros((pad,), jnp.int32)]
    ).reshape(E_PAD // CH, CH)
    row = jnp.concatenate(
        [adj_row.astype(jnp.int32), jnp.zeros((pad,), jnp.int32)])
    val = jnp.concatenate(
        [adj_val.astype(jnp.float32), jnp.zeros((pad,), jnp.float32)])

    y1 = _layer(x, col, row, val)
    y2 = _layer(y1, col, row, val)
    y3 = _layer(y2, col, row, val)
    m = _mean(y1, y2, y3)
    return m[:N_USERS], m[N_USERS:]
